# baseline (speedup 1.0000x reference)
import functools

import jax
import jax.numpy as jnp
from jax import lax
from jax.experimental import pallas as pl
from jax.experimental.pallas import tpu as pltpu
from jax.experimental.pallas import tpu_sc as plsc

N_TOKENS = 16384
N_EMBED = 2048
N_EXPERTS = 64
K_TOP = 8

NC = 2
NS = 16
LANES = 16
NW = NC * NS

N_SPLITS = 4
CHUNK = N_TOKENS // N_SPLITS
TOK_W = CHUNK // NW
N_CHUNKS = TOK_W // LANES

MM_BLK = 1024


def _matmul_body(x_ref, w_ref, b_ref, out_ref):
    out_ref[...] = (
        lax.dot_general(
            w_ref[...], x_ref[...],
            (((1,), (1,)), ((), ())),
            preferred_element_type=jnp.float32,
        )
        + b_ref[...]
    )


def _logits_t(x_chunk, W, b2):
    return pl.pallas_call(
        _matmul_body,
        grid=(CHUNK // MM_BLK,),
        in_specs=[
            pl.BlockSpec((MM_BLK, N_EMBED), lambda i: (i, 0)),
            pl.BlockSpec((N_EXPERTS, N_EMBED), lambda i: (0, 0)),
            pl.BlockSpec((N_EXPERTS, 1), lambda i: (0, 0)),
        ],
        out_specs=pl.BlockSpec((N_EXPERTS, MM_BLK), lambda i: (0, i)),
        out_shape=jax.ShapeDtypeStruct((N_EXPERTS, CHUNK), jnp.float32),
    )(x_chunk, W, b2)


def _topk_body(logits_hbm, idx_hbm, gates_hbm, logits_v, idx_v, gates_v):
    wid = lax.axis_index("s") * NC + lax.axis_index("c")
    base = wid * TOK_W
    pltpu.sync_copy(logits_hbm.at[:, pl.ds(base, TOK_W)], logits_v)

    def chunk_body(c, _):
        off = c * LANES

        def expert_body(e, carry):
            s = list(carry[:K_TOP])
            ids = list(carry[K_TOP:])
            v = logits_v[e, pl.ds(off, LANES)]
            iv = jnp.full((LANES,), e, dtype=jnp.int32)
            for k in range(K_TOP):
                m = v > s[k]
                sv, si = s[k], ids[k]
                s[k] = jnp.where(m, v, sv)
                ids[k] = jnp.where(m, iv, si)
                v = jnp.where(m, sv, v)
                iv = jnp.where(m, si, iv)
            return tuple(s) + tuple(ids)

        neg = jnp.full((LANES,), -jnp.inf, dtype=jnp.float32)
        zero = jnp.zeros((LANES,), dtype=jnp.int32)
        init = (neg,) * K_TOP + (zero,) * K_TOP
        carry = lax.fori_loop(0, N_EXPERTS, expert_body, init)
        s = carry[:K_TOP]
        ids = carry[K_TOP:]

        exps = [jnp.exp(s[k] - s[0]) for k in range(K_TOP)]
        total = exps[0]
        for k in range(1, K_TOP):
            total = total + exps[k]
        inv = jnp.float32(1.0) / total
        for k in range(K_TOP):
            idx_v[k, pl.ds(off, LANES)] = ids[k]
            gates_v[k, pl.ds(off, LANES)] = exps[k] * inv
        return 0

    lax.fori_loop(0, N_CHUNKS, chunk_body, 0)
    pltpu.sync_copy(idx_v, idx_hbm.at[:, pl.ds(base, TOK_W)])
    pltpu.sync_copy(gates_v, gates_hbm.at[:, pl.ds(base, TOK_W)])


@functools.cache
def _topk_sc():
    return functools.partial(
        pl.kernel,
        out_type=(
            jax.ShapeDtypeStruct((K_TOP, CHUNK), jnp.int32),
            jax.ShapeDtypeStruct((K_TOP, CHUNK), jnp.float32),
        ),
        mesh=plsc.VectorSubcoreMesh(core_axis_name="c", subcore_axis_name="s",
                                    num_cores=NC, num_subcores=NS),
        scratch_types=[
            pltpu.VMEM((N_EXPERTS, TOK_W), jnp.float32),
            pltpu.VMEM((K_TOP, TOK_W), jnp.int32),
            pltpu.VMEM((K_TOP, TOK_W), jnp.float32),
        ],
    )(_topk_body)


def kernel(x, W, b):
    b2 = b.reshape(N_EXPERTS, 1)
    topk = _topk_sc()
    idx_parts = []
    gate_parts = []
    for c in range(N_SPLITS):
        x_chunk = lax.slice_in_dim(x, c * CHUNK, (c + 1) * CHUNK, axis=0)
        logits_t = _logits_t(x_chunk, W, b2)
        idx_c, gates_c = topk(logits_t)
        idx_parts.append(idx_c.T)
        gate_parts.append(gates_c.T)
    return (jnp.concatenate(idx_parts, axis=0),
            jnp.concatenate(gate_parts, axis=0))


# speedup vs baseline: 2.1062x; 2.1062x over previous
import functools

import jax
import jax.numpy as jnp
from jax import lax
from jax.experimental import pallas as pl
from jax.experimental.pallas import tpu as pltpu
from jax.experimental.pallas import tpu_sc as plsc

N_TOKENS = 16384
N_EMBED = 2048
N_EXPERTS = 64
K_TOP = 8

NC = 2
NS = 16
LANES = 16
NW = NC * NS

MM_BLK = 1024
SPLITS = (8192, 4096, 4096)


def _matmul_body(x_ref, w_ref, b_ref, out_ref):
    out_ref[...] = (
        lax.dot_general(
            w_ref[...], x_ref[...],
            (((1,), (1,)), ((), ())),
            preferred_element_type=jnp.float32,
        )
        + b_ref[...]
    )


def _logits_t(x, W, b2, start_blk, nblk):
    return pl.pallas_call(
        _matmul_body,
        grid=(nblk,),
        in_specs=[
            pl.BlockSpec((MM_BLK, N_EMBED), lambda i: (start_blk + i, 0)),
            pl.BlockSpec((N_EXPERTS, N_EMBED), lambda i: (0, 0)),
            pl.BlockSpec((N_EXPERTS, 1), lambda i: (0, 0)),
        ],
        out_specs=pl.BlockSpec((N_EXPERTS, MM_BLK), lambda i: (0, i)),
        out_shape=jax.ShapeDtypeStruct((N_EXPERTS, nblk * MM_BLK), jnp.float32),
    )(x, W, b2)


def _make_topk_body(tok_w, n_groups):
    def _topk_body(logits_hbm, idx_hbm, gates_hbm, logits_v, idx_v, gates_v):
        wid = lax.axis_index("s") * NC + lax.axis_index("c")
        base = wid * tok_w
        pltpu.sync_copy(logits_hbm.at[:, pl.ds(base, tok_w)], logits_v)

        def group_body(c, _):
            off = c * LANES

            def expert_body(e, carry):
                s = list(carry[:K_TOP])
                ids = list(carry[K_TOP:])
                v = logits_v[e, pl.ds(off, LANES)]
                iv = jnp.full((LANES,), e, dtype=jnp.int32)
                for k in range(K_TOP):
                    m = v > s[k]
                    sv, si = s[k], ids[k]
                    s[k] = jnp.where(m, v, sv)
                    ids[k] = jnp.where(m, iv, si)
                    v = jnp.where(m, sv, v)
                    iv = jnp.where(m, si, iv)
                return tuple(s) + tuple(ids)

            neg = jnp.full((LANES,), -jnp.inf, dtype=jnp.float32)
            zero = jnp.zeros((LANES,), dtype=jnp.int32)
            init = (neg,) * K_TOP + (zero,) * K_TOP
            carry = lax.fori_loop(0, N_EXPERTS, expert_body, init)
            s = carry[:K_TOP]
            ids = carry[K_TOP:]

            exps = [jnp.exp(s[k] - s[0]) for k in range(K_TOP)]
            total = exps[0]
            for k in range(1, K_TOP):
                total = total + exps[k]
            inv = jnp.float32(1.0) / total
            for k in range(K_TOP):
                idx_v[k, pl.ds(off, LANES)] = ids[k]
                gates_v[k, pl.ds(off, LANES)] = exps[k] * inv
            return 0

        lax.fori_loop(0, n_groups, group_body, 0)
        pltpu.sync_copy(idx_v, idx_hbm.at[:, pl.ds(base, tok_w)])
        pltpu.sync_copy(gates_v, gates_hbm.at[:, pl.ds(base, tok_w)])

    return _topk_body


@functools.cache
def _topk_sc(chunk):
    tok_w = chunk // NW
    n_groups = tok_w // LANES
    return functools.partial(
        pl.kernel,
        out_type=(
            jax.ShapeDtypeStruct((K_TOP, chunk), jnp.int32),
            jax.ShapeDtypeStruct((K_TOP, chunk), jnp.float32),
        ),
        mesh=plsc.VectorSubcoreMesh(core_axis_name="c", subcore_axis_name="s",
                                    num_cores=NC, num_subcores=NS),
        scratch_types=[
            pltpu.VMEM((N_EXPERTS, tok_w), jnp.float32),
            pltpu.VMEM((K_TOP, tok_w), jnp.int32),
            pltpu.VMEM((K_TOP, tok_w), jnp.float32),
        ],
    )(_make_topk_body(tok_w, n_groups))


def kernel(x, W, b):
    b2 = b.reshape(N_EXPERTS, 1)
    idx_parts = []
    gate_parts = []
    off = 0
    for chunk in SPLITS:
        logits_t = _logits_t(x, W, b2, off // MM_BLK, chunk // MM_BLK)
        idx_c, gates_c = _topk_sc(chunk)(logits_t)
        idx_parts.append(idx_c.T)
        gate_parts.append(gates_c.T)
        off += chunk
    return (jnp.concatenate(idx_parts, axis=0),
            jnp.concatenate(gate_parts, axis=0))
